# Initial kernel scaffold; baseline (speedup 1.0000x reference)
import jax
import jax.numpy as jnp
from jax.experimental import pallas as pl


def kernel(x, edge_index, W1, b1, W2, b2, Wd, bd):
    raise NotImplementedError("write your pallas kernel here")


# speedup vs baseline: 9.8008x; 9.8008x over previous
import functools

import jax
import jax.numpy as jnp
from jax import lax
from jax.experimental import pallas as pl
from jax.experimental.pallas import tpu as pltpu
from jax.experimental.pallas import tpu_sc as plsc

N = 10000
E = 320000
D_IN = 165

N_PAD = 10240
TRASH = N
NTILES = 32
CHUNK = 128
CH = -(-E // (NTILES * CHUNK))
E_PAD = CH * NTILES * CHUNK
ROWS_PER_TILE = N_PAD // 16
RB = 256


def _degree_kernel():
    mesh = plsc.VectorSubcoreMesh(core_axis_name="c", subcore_axis_name="s")

    @functools.partial(
        pl.kernel,
        out_type=jax.ShapeDtypeStruct((2, N_PAD, 16), jnp.float32),
        mesh=mesh,
        compiler_params=pltpu.CompilerParams(use_tc_tiling_on_sc=False),
        scratch_types=[
            pltpu.VMEM((CHUNK,), jnp.int32),
            pltpu.VMEM((CHUNK, 16), jnp.float32),
            pltpu.VMEM_SHARED((N_PAD, 16), jnp.float32),
        ],
    )
    def deg(col_hbm, ones_hbm, zeros_hbm, out_hbm, idx_c, buf, acc):
        cid = lax.axis_index("c")
        sid = lax.axis_index("s")
        tile = cid * 16 + sid
        ebase = tile * (CH * CHUNK)

        pltpu.sync_copy(zeros_hbm, buf)
        for j in range(ROWS_PER_TILE // CHUNK):
            pltpu.sync_copy(buf, acc.at[pl.ds(sid * ROWS_PER_TILE + j * CHUNK, CHUNK)])
        plsc.subcore_barrier()

        pltpu.sync_copy(ones_hbm, buf)

        def body(i, carry):
            pltpu.sync_copy(col_hbm.at[pl.ds(ebase + i * CHUNK, CHUNK)], idx_c)
            pltpu.sync_copy(buf, acc.at[idx_c], add=True)
            return carry

        lax.fori_loop(0, CH, body, 0)
        plsc.subcore_barrier()

        for j in range(ROWS_PER_TILE // CHUNK):
            off = sid * ROWS_PER_TILE + j * CHUNK
            pltpu.sync_copy(acc.at[pl.ds(off, CHUNK)], buf)
            pltpu.sync_copy(buf, out_hbm.at[cid, pl.ds(off, CHUNK)])

    return deg


def _mp_kernel(D):
    mesh = plsc.VectorSubcoreMesh(core_axis_name="c", subcore_axis_name="s")

    @functools.partial(
        pl.kernel,
        out_type=jax.ShapeDtypeStruct((2, N_PAD, D), jnp.float32),
        mesh=mesh,
        compiler_params=pltpu.CompilerParams(use_tc_tiling_on_sc=False),
        scratch_types=[
            pltpu.VMEM((CHUNK,), jnp.int32),
            pltpu.VMEM((CHUNK,), jnp.int32),
            pltpu.VMEM((CHUNK, D), jnp.float32),
            pltpu.VMEM_SHARED((N_PAD, D), jnp.float32),
            pltpu.SemaphoreType.DMA,
        ],
    )
    def mp(row_hbm, col_hbm, table_hbm, zeros_hbm, out_hbm,
           idx_r, idx_c, rows, acc, sem):
        cid = lax.axis_index("c")
        sid = lax.axis_index("s")
        tile = cid * 16 + sid
        ebase = tile * (CH * CHUNK)

        pltpu.sync_copy(zeros_hbm, rows)
        for j in range(ROWS_PER_TILE // CHUNK):
            pltpu.sync_copy(rows, acc.at[pl.ds(sid * ROWS_PER_TILE + j * CHUNK, CHUNK)])
        plsc.subcore_barrier()

        def body(i, carry):
            pltpu.sync_copy(row_hbm.at[pl.ds(ebase + i * CHUNK, CHUNK)], idx_r)
            pltpu.sync_copy(col_hbm.at[pl.ds(ebase + i * CHUNK, CHUNK)], idx_c)
            pltpu.async_copy(table_hbm.at[idx_r], rows, sem).wait()
            pltpu.sync_copy(rows, acc.at[idx_c], add=True)
            return carry

        lax.fori_loop(0, CH, body, 0)
        plsc.subcore_barrier()

        for j in range(ROWS_PER_TILE // CHUNK):
            off = sid * ROWS_PER_TILE + j * CHUNK
            pltpu.sync_copy(acc.at[pl.ds(off, CHUNK)], rows)
            pltpu.sync_copy(rows, out_hbm.at[cid, pl.ds(off, CHUNK)])

    return mp


def _dis(degp0, degp1):
    deg = degp0[:, 0:1] + degp1[:, 0:1] + 1.0
    return lax.rsqrt(deg)


def _tc_a(degp_ref, x_ref, w_ref, out_ref):
    dis = _dis(degp_ref[0], degp_ref[1])
    out_ref[...] = dis * jnp.dot(x_ref[...], w_ref[...],
                                 preferred_element_type=jnp.float32)


def _tc_b(degp_ref, acc_ref, tab_ref, b_ref, w_ref, out_ref):
    dis = _dis(degp_ref[0], degp_ref[1])
    s = acc_ref[0] + acc_ref[1] + tab_ref[...]
    h = jnp.maximum(dis * s + b_ref[...], 0.0)
    out_ref[...] = dis * jnp.dot(h, w_ref[...],
                                 preferred_element_type=jnp.float32)


def _tc_c(degp_ref, acc_ref, tab_ref, b_ref, w_ref, bd_ref, out_ref):
    dis = _dis(degp_ref[0], degp_ref[1])
    s = acc_ref[0] + acc_ref[1] + tab_ref[...]
    emb = jnp.maximum(dis * s + b_ref[...], 0.0)
    out_ref[...] = jnp.dot(emb, w_ref[...],
                           preferred_element_type=jnp.float32) + bd_ref[...]


def _row_blocked(d):
    return pl.BlockSpec((RB, d), lambda i: (i, 0))


def _deg_spec():
    return pl.BlockSpec((2, RB, 16), lambda i: (0, i, 0))


def _acc_spec(d):
    return pl.BlockSpec((2, RB, d), lambda i: (0, i, 0))


def _full(shape):
    return pl.BlockSpec(shape, lambda i: tuple(0 for _ in shape))


@jax.jit
def kernel(x, edge_index, W1, b1, W2, b2, Wd, bd):
    f32 = jnp.float32
    row = edge_index[0]
    col = edge_index[1]
    pad = E_PAD - E
    row_p = jnp.concatenate([row, jnp.zeros((pad,), row.dtype)])
    col_p = jnp.concatenate([col, jnp.full((pad,), TRASH, col.dtype)])

    ones16 = jnp.ones((CHUNK, 16), f32)
    zeros16 = jnp.zeros((CHUNK, 16), f32)
    zeros128 = jnp.zeros((CHUNK, 128), f32)
    zeros64 = jnp.zeros((CHUNK, 64), f32)

    degp = _degree_kernel()(col_p, ones16, zeros16)

    k1 = 256
    x_pad = jnp.zeros((N_PAD, k1), f32).at[:N, :D_IN].set(x)
    w1t = jnp.zeros((k1, 128), f32).at[:D_IN, :].set(W1.T)
    grid = (N_PAD // RB,)
    table1 = pl.pallas_call(
        _tc_a,
        grid=grid,
        in_specs=[_deg_spec(), _row_blocked(k1), _full((k1, 128))],
        out_specs=_row_blocked(128),
        out_shape=jax.ShapeDtypeStruct((N_PAD, 128), f32),
    )(degp, x_pad, w1t)

    acc1 = _mp_kernel(128)(row_p, col_p, table1, zeros128)

    table2 = pl.pallas_call(
        _tc_b,
        grid=grid,
        in_specs=[_deg_spec(), _acc_spec(128), _row_blocked(128),
                  _full((1, 128)), _full((128, 64))],
        out_specs=_row_blocked(64),
        out_shape=jax.ShapeDtypeStruct((N_PAD, 64), f32),
    )(degp, acc1, table1, b1.reshape(1, 128), W2.T)

    acc2 = _mp_kernel(64)(row_p, col_p, table2, zeros64)

    dout = 256
    wdt = jnp.zeros((64, dout), f32).at[:, :D_IN].set(Wd.T)
    bd_pad = jnp.zeros((1, dout), f32).at[0, :D_IN].set(bd)
    out = pl.pallas_call(
        _tc_c,
        grid=grid,
        in_specs=[_deg_spec(), _acc_spec(64), _row_blocked(64),
                  _full((1, 64)), _full((64, dout)), _full((1, dout))],
        out_specs=_row_blocked(dout),
        out_shape=jax.ShapeDtypeStruct((N_PAD, dout), f32),
    )(degp, acc2, table2, b2.reshape(1, 64), wdt, bd_pad)

    return out[:N, :D_IN]


# speedup vs baseline: 10.3479x; 1.0558x over previous
import functools

import jax
import jax.numpy as jnp
from jax import lax
from jax.experimental import pallas as pl
from jax.experimental.pallas import tpu as pltpu
from jax.experimental.pallas import tpu_sc as plsc

N = 10000
E = 320000
D_IN = 165

N_PAD = 10240
TRASH = N
NTILES = 32
CHUNK = 128
RC = E // CHUNK
BASE = RC // NTILES
EXTRA = RC - BASE * NTILES
CH = 80
DW = 64
ROWS_PER_TILE = N_PAD // 16
RB = 256
_SC_PARAMS = pltpu.CompilerParams(use_tc_tiling_on_sc=False)


def _load_indices(edges_hbm, pads_hbm, dim, idx, t, pad_row):
    pltpu.sync_copy(edges_hbm.at[dim, pl.ds(t * BASE, BASE)],
                    idx.at[pl.ds(0, BASE)])

    @pl.when(t < EXTRA)
    def _():
        pltpu.sync_copy(edges_hbm.at[dim, pl.ds(BASE * NTILES + t, 1)],
                        idx.at[pl.ds(BASE, 1)])

    @pl.when(t >= EXTRA)
    def _():
        pltpu.sync_copy(pads_hbm.at[pl.ds(pad_row, 1)], idx.at[pl.ds(BASE, 1)])

    for j in range(BASE + 1, CH):
        pltpu.sync_copy(pads_hbm.at[pl.ds(pad_row, 1)], idx.at[pl.ds(j, 1)])


def _degree_kernel():
    mesh = plsc.VectorSubcoreMesh(core_axis_name="c", subcore_axis_name="s")

    @functools.partial(
        pl.kernel,
        out_type=jax.ShapeDtypeStruct((2, N_PAD, 16), jnp.float32),
        mesh=mesh,
        compiler_params=_SC_PARAMS,
        scratch_types=[
            pltpu.VMEM((CH, CHUNK), jnp.int32),
            pltpu.VMEM((CHUNK, 16), jnp.float32),
            pltpu.VMEM_SHARED((N_PAD, 16), jnp.float32),
            pltpu.SemaphoreType.DMA,
            pltpu.SemaphoreType.DMA,
            pltpu.SemaphoreType.DMA,
            pltpu.SemaphoreType.DMA,
        ],
    )
    def deg(edges_hbm, ones_hbm, zeros_hbm, pads_hbm, out_hbm,
            idx_c, buf, acc, s0, s1, s2, s3):
        sems = [s0, s1, s2, s3]
        cid = lax.axis_index("c")
        sid = lax.axis_index("s")
        t = cid * 16 + sid

        pltpu.sync_copy(zeros_hbm, buf)
        for j in range(ROWS_PER_TILE // CHUNK):
            pltpu.sync_copy(buf, acc.at[pl.ds(sid * ROWS_PER_TILE + j * CHUNK, CHUNK)])
        _load_indices(edges_hbm, pads_hbm, 1, idx_c, t, 1)
        pltpu.sync_copy(ones_hbm, buf)
        plsc.subcore_barrier()

        def issue(i, k):
            pltpu.async_copy(buf, acc.at[idx_c.at[i]], sems[k], add=True)

        def drain(k):
            pltpu.make_async_copy(buf, acc.at[idx_c.at[0]], sems[k]).wait()

        for k in range(4):
            issue(k, k)

        def body(s, carry):
            for k in range(4):
                drain(k)
                issue(4 * s + k, k)
            return carry

        lax.fori_loop(1, CH // 4, body, 0)
        for k in range(4):
            drain(k)
        plsc.subcore_barrier()

        for j in range(ROWS_PER_TILE // CHUNK):
            off = sid * ROWS_PER_TILE + j * CHUNK
            pltpu.sync_copy(acc.at[pl.ds(off, CHUNK)], buf)
            pltpu.sync_copy(buf, out_hbm.at[cid, pl.ds(off, CHUNK)])

    return deg


def _mp_kernel(nt):
    mesh = plsc.VectorSubcoreMesh(core_axis_name="c", subcore_axis_name="s")

    @functools.partial(
        pl.kernel,
        out_type=jax.ShapeDtypeStruct((2, nt, N_PAD, DW), jnp.float32),
        mesh=mesh,
        compiler_params=_SC_PARAMS,
        scratch_types=[
            pltpu.VMEM((CH, CHUNK), jnp.int32),
            pltpu.VMEM((CH, CHUNK), jnp.int32),
        ] + [pltpu.VMEM((CHUNK, DW), jnp.float32) for _ in range(6)]
          + [pltpu.VMEM_SHARED((N_PAD, DW), jnp.float32)]
          + [pltpu.SemaphoreType.DMA for _ in range(10)],
    )
    def mp(edges_hbm, *rest):
        tables = rest[:nt]
        zeros_hbm, pads_hbm, out_hbm = rest[nt:nt + 3]
        idx_r, idx_c, r0, r1, r2, r3, r4, zbuf, acc = rest[nt + 3:nt + 12]
        semg = rest[nt + 12:nt + 17]
        sems = rest[nt + 17:nt + 22]
        rows = [r0, r1, r2, r3, r4]
        cid = lax.axis_index("c")
        sid = lax.axis_index("s")
        t = cid * 16 + sid

        _load_indices(edges_hbm, pads_hbm, 0, idx_r, t, 0)
        _load_indices(edges_hbm, pads_hbm, 1, idx_c, t, 1)
        pltpu.sync_copy(zeros_hbm, zbuf)

        for h in range(nt):
            table_hbm = tables[h]
            for j in range(ROWS_PER_TILE // CHUNK):
                pltpu.sync_copy(
                    zbuf, acc.at[pl.ds(sid * ROWS_PER_TILE + j * CHUNK, CHUNK)])
            plsc.subcore_barrier()

            def issue_gather(i, b):
                pltpu.async_copy(table_hbm.at[idx_r.at[i]], rows[b], semg[b])

            def wait_gather(b):
                pltpu.make_async_copy(table_hbm.at[idx_r.at[0]], rows[b],
                                      semg[b]).wait()

            def issue_scatter(i, b):
                pltpu.async_copy(rows[b], acc.at[idx_c.at[i]], sems[b],
                                 add=True)

            def wait_scatter(b):
                pltpu.make_async_copy(rows[b], acc.at[idx_c.at[0]],
                                      sems[b]).wait()

            for b in range(3):
                issue_gather(b, b)
            wait_gather(0)
            issue_scatter(0, 0)
            issue_gather(3, 3)
            wait_gather(1)
            issue_scatter(1, 1)
            issue_gather(4, 4)

            def body(s, carry):
                i0 = 5 * s + 2
                for k in range(5):
                    b = (2 + k) % 5
                    wait_gather(b)
                    issue_scatter(i0 + k, b)
                    wait_scatter(k)
                    issue_gather(i0 + k + 3, (b + 3) % 5)
                return carry

            lax.fori_loop(0, 15, body, 0)

            for i, b in ((77, 2), (78, 3), (79, 4)):
                wait_gather(b)
                issue_scatter(i, b)
            for b in range(5):
                wait_scatter(b)
            plsc.subcore_barrier()

            for j in range(ROWS_PER_TILE // CHUNK):
                off = sid * ROWS_PER_TILE + j * CHUNK
                pltpu.sync_copy(acc.at[pl.ds(off, CHUNK)], r1)
                pltpu.sync_copy(r1, out_hbm.at[cid, h, pl.ds(off, CHUNK)])

    return mp


def _dis(degp0, degp1):
    deg = degp0[:, 0:1] + degp1[:, 0:1] + 1.0
    return lax.rsqrt(deg)


def _tc_a0(x_ref, w_ref, out_ref):
    out_ref[...] = jnp.dot(x_ref[...], w_ref[...],
                           preferred_element_type=jnp.float32)


def _tc_a1(degp_ref, xw_ref, outa_ref, outb_ref):
    dis = _dis(degp_ref[0], degp_ref[1])
    xw = xw_ref[...]
    outa_ref[...] = dis * xw[:, :DW]
    outb_ref[...] = dis * xw[:, DW:]


def _tc_b(degp_ref, acc_ref, taba_ref, tabb_ref, b_ref, w_ref, out_ref):
    dis = _dis(degp_ref[0], degp_ref[1])
    b = b_ref[...]
    w = w_ref[...]
    sa = acc_ref[0, 0] + acc_ref[1, 0] + taba_ref[...]
    sb = acc_ref[0, 1] + acc_ref[1, 1] + tabb_ref[...]
    ha = jnp.maximum(dis * sa + b[:, :DW], 0.0)
    hb = jnp.maximum(dis * sb + b[:, DW:], 0.0)
    out_ref[...] = dis * (
        jnp.dot(ha, w[:DW], preferred_element_type=jnp.float32)
        + jnp.dot(hb, w[DW:], preferred_element_type=jnp.float32))


def _tc_c(degp_ref, acc_ref, tab_ref, b_ref, w_ref, bd_ref, out_ref):
    dis = _dis(degp_ref[0], degp_ref[1])
    s = acc_ref[0, 0] + acc_ref[1, 0] + tab_ref[...]
    emb = jnp.maximum(dis * s + b_ref[...], 0.0)
    out_ref[...] = jnp.dot(emb, w_ref[...],
                           preferred_element_type=jnp.float32) + bd_ref[...]


def _row_blocked(d):
    return pl.BlockSpec((RB, d), lambda i: (i, 0))


def _deg_spec():
    return pl.BlockSpec((2, RB, 16), lambda i: (0, i, 0))


def _acc_spec(nt):
    return pl.BlockSpec((2, nt, RB, DW), lambda i: (0, 0, i, 0))


def _full(shape):
    return pl.BlockSpec(shape, lambda i: tuple(0 for _ in shape))


@jax.jit
def kernel(x, edge_index, W1, b1, W2, b2, Wd, bd):
    f32 = jnp.float32
    i32 = jnp.int32
    edges3 = edge_index.reshape(2, RC, CHUNK)
    pads = jnp.stack([jnp.zeros((CHUNK,), i32),
                      jnp.full((CHUNK,), TRASH, i32)])

    ones16 = jnp.ones((CHUNK, 16), f32)
    zeros16 = jnp.zeros((CHUNK, 16), f32)
    zerosw = jnp.zeros((CHUNK, DW), f32)

    degp = _degree_kernel()(edges3, ones16, zeros16, pads)

    k1 = 256
    x_pad = jnp.zeros((N_PAD, k1), f32).at[:N, :D_IN].set(x)
    w1t = jnp.zeros((k1, 128), f32).at[:D_IN, :].set(W1.T)
    grid = (N_PAD // RB,)
    xw1 = pl.pallas_call(
        _tc_a0,
        grid=grid,
        in_specs=[_row_blocked(k1), _full((k1, 128))],
        out_specs=_row_blocked(128),
        out_shape=jax.ShapeDtypeStruct((N_PAD, 128), f32),
    )(x_pad, w1t)
    table1a, table1b = pl.pallas_call(
        _tc_a1,
        grid=grid,
        in_specs=[_deg_spec(), _row_blocked(128)],
        out_specs=[_row_blocked(DW), _row_blocked(DW)],
        out_shape=[jax.ShapeDtypeStruct((N_PAD, DW), f32)] * 2,
    )(degp, xw1)

    acc1 = _mp_kernel(2)(edges3, table1a, table1b, zerosw, pads)

    table2 = pl.pallas_call(
        _tc_b,
        grid=grid,
        in_specs=[_deg_spec(), _acc_spec(2), _row_blocked(DW),
                  _row_blocked(DW), _full((1, 128)), _full((128, DW))],
        out_specs=_row_blocked(DW),
        out_shape=jax.ShapeDtypeStruct((N_PAD, DW), f32),
    )(degp, acc1, table1a, table1b, b1.reshape(1, 128), W2.T)

    acc2 = _mp_kernel(1)(edges3, table2, zerosw, pads)

    dout = 256
    wdt = jnp.zeros((DW, dout), f32).at[:, :D_IN].set(Wd.T)
    bd_pad = jnp.zeros((1, dout), f32).at[0, :D_IN].set(bd)
    out = pl.pallas_call(
        _tc_c,
        grid=grid,
        in_specs=[_deg_spec(), _acc_spec(1), _row_blocked(DW),
                  _full((1, DW)), _full((DW, dout)), _full((1, dout))],
        out_specs=_row_blocked(dout),
        out_shape=jax.ShapeDtypeStruct((N_PAD, dout), f32),
    )(degp, acc2, table2, b2.reshape(1, DW), wdt, bd_pad)

    return out[:N, :D_IN]


# speedup vs baseline: 11.3709x; 1.0989x over previous
import functools

import jax
import jax.numpy as jnp
from jax import lax
from jax.experimental import pallas as pl
from jax.experimental.pallas import tpu as pltpu
from jax.experimental.pallas import tpu_sc as plsc

N = 10000
E = 320000
D_IN = 165

N_PAD = 10240
TRASH = N
NTILES = 32
CHUNK = 128
RC = E // CHUNK
BASE = RC // NTILES
EXTRA = RC - BASE * NTILES
CH = 80
ROWS_PER_TILE = N_PAD // 16
RB = 400
_SC_PARAMS = pltpu.CompilerParams(use_tc_tiling_on_sc=False)


def _degree_kernel():
    mesh = plsc.VectorSubcoreMesh(core_axis_name="c", subcore_axis_name="s")

    @functools.partial(
        pl.kernel,
        out_type=jax.ShapeDtypeStruct((2, N_PAD, 16), jnp.float32),
        mesh=mesh,
        compiler_params=_SC_PARAMS,
        scratch_types=[
            pltpu.VMEM((CH, CHUNK), jnp.int32),
            pltpu.VMEM((CHUNK, 16), jnp.float32),
            pltpu.VMEM_SHARED((N_PAD, 16), jnp.float32),
            pltpu.SemaphoreType.DMA,
            pltpu.SemaphoreType.DMA,
            pltpu.SemaphoreType.DMA,
            pltpu.SemaphoreType.DMA,
        ],
    )
    def deg(edges_hbm, ones_hbm, zeros_hbm, pads_hbm, out_hbm,
            idx_c, buf, acc, s0, s1, s2, s3):
        sems = [s0, s1, s2, s3]
        cid = lax.axis_index("c")
        sid = lax.axis_index("s")
        t = cid * 16 + sid

        pltpu.sync_copy(zeros_hbm, buf)
        for j in range(ROWS_PER_TILE // CHUNK):
            pltpu.sync_copy(buf, acc.at[pl.ds(sid * ROWS_PER_TILE + j * CHUNK, CHUNK)])

        pltpu.sync_copy(edges_hbm.at[1, pl.ds(t * BASE, BASE)],
                        idx_c.at[pl.ds(0, BASE)])

        @pl.when(t < EXTRA)
        def _():
            pltpu.sync_copy(edges_hbm.at[1, pl.ds(BASE * NTILES + t, 1)],
                            idx_c.at[pl.ds(BASE, 1)])

        @pl.when(t >= EXTRA)
        def _():
            pltpu.sync_copy(pads_hbm, idx_c.at[pl.ds(BASE, 1)])

        for j in range(BASE + 1, CH):
            pltpu.sync_copy(pads_hbm, idx_c.at[pl.ds(j, 1)])

        pltpu.sync_copy(ones_hbm, buf)
        plsc.subcore_barrier()

        def issue(i, k):
            pltpu.async_copy(buf, acc.at[idx_c.at[i]], sems[k], add=True)

        def drain(k):
            pltpu.make_async_copy(buf, acc.at[idx_c.at[0]], sems[k]).wait()

        for k in range(4):
            issue(k, k)

        def body(s, carry):
            for k in range(4):
                drain(k)
                issue(4 * s + k, k)
            return carry

        lax.fori_loop(1, CH // 4, body, 0)
        for k in range(4):
            drain(k)
        plsc.subcore_barrier()

        for j in range(ROWS_PER_TILE // CHUNK):
            off = sid * ROWS_PER_TILE + j * CHUNK
            pltpu.sync_copy(acc.at[pl.ds(off, CHUNK)], buf)
            pltpu.sync_copy(buf, out_hbm.at[cid, pl.ds(off, CHUNK)])

    return deg


def _mp_kernel(D):
    mesh = plsc.VectorSubcoreMesh(core_axis_name="c", subcore_axis_name="s")

    @functools.partial(
        pl.kernel,
        out_type=jax.ShapeDtypeStruct((2, N_PAD, D), jnp.float32),
        mesh=mesh,
        compiler_params=_SC_PARAMS,
        scratch_types=[
            pltpu.VMEM((CH * CHUNK,), jnp.int32),
            pltpu.VMEM((2, CHUNK), jnp.int32),
            pltpu.VMEM((2, CHUNK), jnp.int32),
            pltpu.VMEM((CHUNK, D), jnp.float32),
            pltpu.VMEM((CHUNK, D), jnp.float32),
            pltpu.VMEM_SHARED((N_PAD, D), jnp.float32),
            pltpu.SemaphoreType.DMA,
            pltpu.SemaphoreType.DMA,
            pltpu.SemaphoreType.DMA,
            pltpu.SemaphoreType.DMA,
        ],
    )
    def mp(packed_hbm, table_hbm, zeros_hbm, pads_hbm, out_hbm,
           idx_p, st_r, st_c, r0, r1, acc, g0, g1, t0, t1):
        rows = [r0, r1]
        semg = [g0, g1]
        sems = [t0, t1]
        cid = lax.axis_index("c")
        sid = lax.axis_index("s")
        t = cid * 16 + sid

        pltpu.sync_copy(packed_hbm.at[pl.ds(t * BASE * CHUNK, BASE * CHUNK)],
                        idx_p.at[pl.ds(0, BASE * CHUNK)])

        @pl.when(t < EXTRA)
        def _():
            pltpu.sync_copy(
                packed_hbm.at[pl.ds((BASE * NTILES + t) * CHUNK, CHUNK)],
                idx_p.at[pl.ds(BASE * CHUNK, CHUNK)])

        @pl.when(t >= EXTRA)
        def _():
            pltpu.sync_copy(pads_hbm, idx_p.at[pl.ds(BASE * CHUNK, CHUNK)])

        for j in range(BASE + 1, CH):
            pltpu.sync_copy(pads_hbm, idx_p.at[pl.ds(j * CHUNK, CHUNK)])

        pltpu.sync_copy(zeros_hbm, r0)
        for j in range(ROWS_PER_TILE // CHUNK):
            pltpu.sync_copy(r0, acc.at[pl.ds(sid * ROWS_PER_TILE + j * CHUNK, CHUNK)])
        plsc.subcore_barrier()

        def unpack(i, b):
            for j in range(CHUNK // 16):
                v = idx_p[pl.ds(i * CHUNK + j * 16, 16)]
                st_r[b, pl.ds(j * 16, 16)] = jnp.bitwise_and(v, 0xFFFF)
                st_c[b, pl.ds(j * 16, 16)] = jnp.right_shift(v, 16)

        def issue_gather(b):
            pltpu.async_copy(table_hbm.at[st_r.at[b]], rows[b], semg[b])

        def wait_gather(b):
            pltpu.make_async_copy(table_hbm.at[st_r.at[b]], rows[b],
                                  semg[b]).wait()

        def issue_scatter(b):
            pltpu.async_copy(rows[b], acc.at[st_c.at[b]], sems[b], add=True)

        def wait_scatter(b):
            pltpu.make_async_copy(rows[b], acc.at[st_c.at[b]], sems[b]).wait()

        unpack(0, 0)
        issue_gather(0)
        unpack(1, 1)
        issue_gather(1)

        def body(s, carry):
            for k in range(2):
                i = 2 * s + k
                wait_gather(k)
                issue_scatter(k)
                wait_scatter(k)
                unpack(i + 2, k)
                issue_gather(k)
            return carry

        lax.fori_loop(0, (CH - 2) // 2, body, 0)

        for k in range(2):
            wait_gather(k)
            issue_scatter(k)
        for k in range(2):
            wait_scatter(k)
        plsc.subcore_barrier()

        for j in range(ROWS_PER_TILE // CHUNK):
            off = sid * ROWS_PER_TILE + j * CHUNK
            pltpu.sync_copy(acc.at[pl.ds(off, CHUNK)], r0)
            pltpu.sync_copy(r0, out_hbm.at[cid, pl.ds(off, CHUNK)])

    return mp


def _dis(degp0, degp1):
    deg = degp0[:, 0:1] + degp1[:, 0:1] + 1.0
    return lax.rsqrt(deg)


def _tc_a0(x_ref, w_ref, out_ref):
    out_ref[...] = jnp.dot(x_ref[...], w_ref[...],
                           preferred_element_type=jnp.float32)


def _tc_a1(degp_ref, xw_ref, out_ref):
    out_ref[...] = _dis(degp_ref[0], degp_ref[1]) * xw_ref[...]


def _tc_b(degp_ref, acc_ref, tab_ref, b_ref, w_ref, out_ref):
    dis = _dis(degp_ref[0], degp_ref[1])
    s = acc_ref[0] + acc_ref[1] + tab_ref[...]
    h = jnp.maximum(dis * s + b_ref[...], 0.0)
    out_ref[...] = dis * jnp.dot(h, w_ref[...],
                                 preferred_element_type=jnp.float32)


def _tc_c(degp_ref, acc_ref, tab_ref, b_ref, w_ref, bd_ref, out_ref):
    dis = _dis(degp_ref[0], degp_ref[1])
    s = acc_ref[0] + acc_ref[1] + tab_ref[...]
    emb = jnp.maximum(dis * s + b_ref[...], 0.0)
    out_ref[...] = jnp.dot(emb, w_ref[...],
                           preferred_element_type=jnp.float32) + bd_ref[...]


def _row_blocked(d):
    return pl.BlockSpec((RB, d), lambda i: (i, 0))


def _deg_spec():
    return pl.BlockSpec((2, RB, 16), lambda i: (0, i, 0))


def _acc_spec(d):
    return pl.BlockSpec((2, RB, d), lambda i: (0, i, 0))


def _full(shape):
    return pl.BlockSpec(shape, lambda i: tuple(0 for _ in shape))


@jax.jit
def kernel(x, edge_index, W1, b1, W2, b2, Wd, bd):
    f32 = jnp.float32
    i32 = jnp.int32
    edges3 = edge_index.reshape(2, RC, CHUNK)
    packed = (edge_index[0] + edge_index[1] * 65536).reshape(E)
    pads_c = jnp.full((1, CHUNK), TRASH, i32)
    pads_p = jnp.full((CHUNK,), TRASH * 65536, i32)

    ones16 = jnp.ones((CHUNK, 16), f32)
    zeros16 = jnp.zeros((CHUNK, 16), f32)
    zeros128 = jnp.zeros((CHUNK, 128), f32)
    zeros64 = jnp.zeros((CHUNK, 64), f32)

    degp = _degree_kernel()(edges3, ones16, zeros16, pads_c)

    grid = (N // RB,)
    xw1 = pl.pallas_call(
        _tc_a0,
        grid=grid,
        in_specs=[_row_blocked(D_IN), _full((D_IN, 128))],
        out_specs=_row_blocked(128),
        out_shape=jax.ShapeDtypeStruct((N, 128), f32),
    )(x, W1.T)
    table1 = pl.pallas_call(
        _tc_a1,
        grid=grid,
        in_specs=[_deg_spec(), _row_blocked(128)],
        out_specs=_row_blocked(128),
        out_shape=jax.ShapeDtypeStruct((N, 128), f32),
    )(degp, xw1)

    acc1 = _mp_kernel(128)(packed, table1, zeros128, pads_p)

    table2 = pl.pallas_call(
        _tc_b,
        grid=grid,
        in_specs=[_deg_spec(), _acc_spec(128), _row_blocked(128),
                  _full((1, 128)), _full((128, 64))],
        out_specs=_row_blocked(64),
        out_shape=jax.ShapeDtypeStruct((N, 64), f32),
    )(degp, acc1, table1, b1.reshape(1, 128), W2.T)

    acc2 = _mp_kernel(64)(packed, table2, zeros64, pads_p)

    dout = 256
    wdt = jnp.zeros((64, dout), f32).at[:, :D_IN].set(Wd.T)
    bd_pad = jnp.zeros((1, dout), f32).at[0, :D_IN].set(bd)
    out = pl.pallas_call(
        _tc_c,
        grid=grid,
        in_specs=[_deg_spec(), _acc_spec(64), _row_blocked(64),
                  _full((1, 64)), _full((64, dout)), _full((1, dout))],
        out_specs=_row_blocked(dout),
        out_shape=jax.ShapeDtypeStruct((N, dout), f32),
    )(degp, acc2, table2, b2.reshape(1, 64), wdt, bd_pad)

    return out[:, :D_IN]
